# baseline (speedup 1.0000x reference)
import functools

import jax
import jax.numpy as jnp
from jax import lax
from jax.experimental import pallas as pl
from jax.experimental.pallas import tpu as pltpu
from jax.experimental.pallas import tpu_sc as plsc

B, N, K = 2, 2048, 16
D = 1024
H = 16
HD = 64
DIM = H * HD

ROW_BLK = 512
NP = N // ROW_BLK
I_BLK = 256
NI = N // I_BLK
SCALE = HD ** -0.5
LOG2E = 1.4426950408889634

NW = 32
RPW = B * N // NW
CH = 16


def _counts_sc_kernel(idx_hbm, zeros_hbm, counts_hbm, idx_v, buf0, buf1,
                      sem0, sem1):
    wid = lax.axis_index("s") * 2 + lax.axis_index("c")
    base = wid * RPW
    pltpu.sync_copy(idx_hbm.at[pl.ds(base * K, RPW * K)], idx_v)
    pltpu.sync_copy(zeros_hbm, buf0)
    pltpu.sync_copy(zeros_hbm, buf1)
    ones = jnp.ones((K,), jnp.float32)
    zros = jnp.zeros((K,), jnp.float32)
    bufs = (buf0, buf1)
    sems = (sem0, sem1)

    def _off(r, col):
        tile = jax.lax.shift_right_logical(col, 7)
        lane = jax.lax.bitwise_and(col, 127)
        return tile * 1024 + lane + ((r // 8) * 16384 + (r % 8) * 128)

    nch = RPW // CH
    pending = [None, None]
    for c in range(nch):
        bi = c % 2
        buf = bufs[bi]
        if pending[bi] is not None:
            pending[bi].wait()
            pc = c - 2
            for r in range(CH):
                col = idx_v[pl.ds((pc * CH + r) * K, K)]
                plsc.store_scatter(buf, [_off(r, col)], zros)
        for r in range(CH):
            col = idx_v[pl.ds((c * CH + r) * K, K)]
            plsc.addupdate_scatter(buf, [_off(r, col)], ones)
        pending[bi] = pltpu.async_copy(
            buf, counts_hbm.at[pl.ds((base + c * CH) * N, CH * N)],
            sems[bi])
    pending[(nch - 1) % 2].wait()
    pending[nch % 2].wait()


def _build_counts(idx_flat):
    mesh = plsc.VectorSubcoreMesh(core_axis_name="c", subcore_axis_name="s")
    zeros = jnp.zeros((CH * N,), jnp.float32)
    f = functools.partial(
        pl.kernel,
        mesh=mesh,
        out_type=jax.ShapeDtypeStruct((B * N * N,), jnp.float32),
        scratch_types=[
            pltpu.VMEM((RPW * K,), jnp.int32),
            pltpu.VMEM((CH * N,), jnp.float32),
            pltpu.VMEM((CH * N,), jnp.float32),
            pltpu.SemaphoreType.DMA,
            pltpu.SemaphoreType.DMA,
        ],
        compiler_params=pltpu.CompilerParams(needs_layout_passes=False),
    )(_counts_sc_kernel)
    return f(idx_flat, zeros).reshape(B * N // 8, N // 128, 8, 128)


def _fused_kernel(ctx_ref, x_ref, cnt_ref, wq_ref, bq_ref, wk_ref, wv_ref,
                  wo_ref, bo_ref, o_ref, k_scr, v_scr):
    s = pl.program_id(1)

    @pl.when(s < NP)
    def _kv_phase():
        c = ctx_ref[0]
        row0 = s * ROW_BLK
        k_scr[pl.ds(row0, ROW_BLK), :] = jnp.dot(
            c, wk_ref[...], preferred_element_type=jnp.float32)
        v_scr[pl.ds(row0, ROW_BLK), :] = jnp.dot(
            c, wv_ref[...], preferred_element_type=jnp.float32)

    @pl.when(s >= NP)
    def _attn_phase():
        xb = x_ref[0]
        q = jnp.dot(xb, wq_ref[...], preferred_element_type=jnp.float32)
        q = (q + bq_ref[...]) * (SCALE * LOG2E)

        cols = [cnt_ref[:, tc].reshape(I_BLK, 128)
                for tc in range(N // 128)]
        counts = jnp.concatenate(cols, axis=1)
        lc = jnp.log2(counts)

        parts = []
        for h in range(H):
            qh = q[:, h * HD:(h + 1) * HD]
            kh = k_scr[:, h * HD:(h + 1) * HD]
            t = jax.lax.dot_general(qh, kh, (((1,), (1,)), ((), ())),
                                    preferred_element_type=jnp.float32)
            t = t + lc
            m = jnp.max(t, axis=1, keepdims=True)
            p = jnp.exp2(t - m)
            denom = jnp.sum(p, axis=1, keepdims=True)
            vh = v_scr[:, h * HD:(h + 1) * HD]
            oh = jax.lax.dot_general(p, vh, (((1,), (0,)), ((), ())),
                                     preferred_element_type=jnp.float32)
            parts.append(oh * (1.0 / denom))
        attn = jnp.concatenate(parts, axis=1)
        o_ref[0] = (jnp.dot(attn, wo_ref[...],
                            preferred_element_type=jnp.float32)
                    + bo_ref[...])


def kernel(x, context, index_pairs, Wq, bq, Wk, Wv, Wo, bo):
    idx_flat = index_pairs.astype(jnp.int32).reshape(B * N * K)
    counts = _build_counts(idx_flat)

    out = pl.pallas_call(
        _fused_kernel,
        grid=(B, NP + NI),
        in_specs=[
            pl.BlockSpec((1, ROW_BLK, D),
                         lambda b, s: (b, jnp.minimum(s, NP - 1), 0)),
            pl.BlockSpec((1, I_BLK, D),
                         lambda b, s: (b, jnp.maximum(s - NP, 0), 0)),
            pl.BlockSpec((I_BLK // 8, N // 128, 8, 128),
                         lambda b, s: (b * NI + jnp.maximum(s - NP, 0),
                                       0, 0, 0)),
            pl.BlockSpec((D, DIM), lambda b, s: (0, 0)),
            pl.BlockSpec((1, DIM), lambda b, s: (0, 0)),
            pl.BlockSpec((D, DIM), lambda b, s: (0, 0)),
            pl.BlockSpec((D, DIM), lambda b, s: (0, 0)),
            pl.BlockSpec((DIM, D), lambda b, s: (0, 0)),
            pl.BlockSpec((1, D), lambda b, s: (0, 0)),
        ],
        out_specs=pl.BlockSpec((1, I_BLK, D),
                               lambda b, s: (b, jnp.maximum(s - NP, 0), 0)),
        out_shape=jax.ShapeDtypeStruct((B, N, D), jnp.float32),
        scratch_shapes=[
            pltpu.VMEM((N, DIM), jnp.float32),
            pltpu.VMEM((N, DIM), jnp.float32),
        ],
    )(context, x, counts, Wq, bq.reshape(1, DIM), Wk, Wv, Wo,
      bo.reshape(1, D))
    return out


# speedup vs baseline: 1.0465x; 1.0465x over previous
import functools

import jax
import jax.numpy as jnp
from jax import lax
from jax.experimental import pallas as pl
from jax.experimental.pallas import tpu as pltpu
from jax.experimental.pallas import tpu_sc as plsc

B, N, K = 2, 2048, 16
D = 1024
H = 16
HD = 64
DIM = H * HD

ROW_BLK = 512
I_BLK = 256
SCALE = HD ** -0.5
LOG2E = 1.4426950408889634

NW = 32
RPW = B * N // NW
CH = 16


def _counts_sc_kernel(idx_hbm, zeros_hbm, counts_hbm, idx_v, buf, sem):
    wid = lax.axis_index("s") * 2 + lax.axis_index("c")
    base = wid * RPW
    pltpu.sync_copy(idx_hbm.at[pl.ds(base * K, RPW * K)], idx_v)
    pltpu.sync_copy(zeros_hbm, buf)
    ones = jnp.ones((K,), jnp.float32)
    zros = jnp.zeros((K,), jnp.float32)

    def _off(r, col):
        tile = jax.lax.shift_right_logical(col, 7)
        lane = jax.lax.bitwise_and(col, 127)
        return tile * 1024 + lane + ((r // 8) * 16384 + (r % 8) * 128)

    for c in range(RPW // CH):
        for r in range(CH):
            col = idx_v[pl.ds((c * CH + r) * K, K)]
            plsc.addupdate_scatter(buf, [_off(r, col)], ones)
        cp = pltpu.async_copy(
            buf, counts_hbm.at[pl.ds((base + c * CH) * N, CH * N)], sem)
        cp.wait()
        if c + 1 < RPW // CH:
            for r in range(CH):
                col = idx_v[pl.ds((c * CH + r) * K, K)]
                plsc.store_scatter(buf, [_off(r, col)], zros)


def _build_counts(idx_flat):
    mesh = plsc.VectorSubcoreMesh(core_axis_name="c", subcore_axis_name="s")
    zeros = jnp.zeros((CH * N,), jnp.float32)
    f = functools.partial(
        pl.kernel,
        mesh=mesh,
        out_type=jax.ShapeDtypeStruct((B * N * N,), jnp.float32),
        scratch_types=[
            pltpu.VMEM((RPW * K,), jnp.int32),
            pltpu.VMEM((CH * N,), jnp.float32),
            pltpu.SemaphoreType.DMA,
        ],
        compiler_params=pltpu.CompilerParams(needs_layout_passes=False),
    )(_counts_sc_kernel)
    return f(idx_flat, zeros).reshape(B * N // 8, N // 128, 8, 128)


def _qkv_proj_kernel(x_ref, ctx_ref, wq_ref, bq_ref, wk_ref, wv_ref,
                     q_ref, k_ref, v_ref):
    xb = x_ref[...]
    c = ctx_ref[...]
    q = jnp.dot(xb, wq_ref[...], preferred_element_type=jnp.float32)
    q_ref[...] = (q + bq_ref[...]) * (SCALE * LOG2E)
    k_ref[...] = jnp.dot(c, wk_ref[...], preferred_element_type=jnp.float32)
    v_ref[...] = jnp.dot(c, wv_ref[...], preferred_element_type=jnp.float32)


def _attn_kernel(q_ref, cnt_ref, k_ref, v_ref, wo_ref, bo_ref, o_ref):
    q = q_ref[0]
    cols = [cnt_ref[:, tc].reshape(I_BLK, 128) for tc in range(N // 128)]
    counts = jnp.concatenate(cols, axis=1)
    lc = jnp.log2(counts)

    parts = []
    for h in range(H):
        qh = q[:, h * HD:(h + 1) * HD]
        kh = k_ref[0, :, h * HD:(h + 1) * HD]
        t = jax.lax.dot_general(qh, kh, (((1,), (1,)), ((), ())),
                                preferred_element_type=jnp.float32)
        t = t + lc
        m = jnp.max(t, axis=1, keepdims=True)
        p = jnp.exp2(t - m)
        denom = jnp.sum(p, axis=1, keepdims=True)
        vh = v_ref[0, :, h * HD:(h + 1) * HD]
        oh = jax.lax.dot_general(p, vh, (((1,), (0,)), ((), ())),
                                 preferred_element_type=jnp.float32)
        parts.append(oh * (1.0 / denom))
    attn = jnp.concatenate(parts, axis=1)
    o_ref[0] = (jnp.dot(attn, wo_ref[...], preferred_element_type=jnp.float32)
                + bo_ref[...])


def kernel(x, context, index_pairs, Wq, bq, Wk, Wv, Wo, bo):
    x2 = x.reshape(B * N, D)
    ctx2 = context.reshape(B * N, D)
    idx_flat = index_pairs.astype(jnp.int32).reshape(B * N * K)

    counts = _build_counts(idx_flat)

    qkv = pl.pallas_call(
        _qkv_proj_kernel,
        grid=(B * N // ROW_BLK,),
        in_specs=[
            pl.BlockSpec((ROW_BLK, D), lambda r: (r, 0)),
            pl.BlockSpec((ROW_BLK, D), lambda r: (r, 0)),
            pl.BlockSpec((D, DIM), lambda r: (0, 0)),
            pl.BlockSpec((1, DIM), lambda r: (0, 0)),
            pl.BlockSpec((D, DIM), lambda r: (0, 0)),
            pl.BlockSpec((D, DIM), lambda r: (0, 0)),
        ],
        out_specs=[
            pl.BlockSpec((ROW_BLK, DIM), lambda r: (r, 0)),
            pl.BlockSpec((ROW_BLK, DIM), lambda r: (r, 0)),
            pl.BlockSpec((ROW_BLK, DIM), lambda r: (r, 0)),
        ],
        out_shape=[
            jax.ShapeDtypeStruct((B * N, DIM), jnp.float32),
            jax.ShapeDtypeStruct((B * N, DIM), jnp.float32),
            jax.ShapeDtypeStruct((B * N, DIM), jnp.float32),
        ],
    )(x2, ctx2, Wq, bq.reshape(1, DIM), Wk, Wv)
    q3 = qkv[0].reshape(B, N, DIM)
    k3 = qkv[1].reshape(B, N, DIM)
    v3 = qkv[2].reshape(B, N, DIM)

    nI = N // I_BLK
    out = pl.pallas_call(
        _attn_kernel,
        grid=(B, nI),
        in_specs=[
            pl.BlockSpec((1, I_BLK, DIM), lambda b, i: (b, i, 0)),
            pl.BlockSpec((I_BLK // 8, N // 128, 8, 128),
                         lambda b, i: (b * nI + i, 0, 0, 0)),
            pl.BlockSpec((1, N, DIM), lambda b, i: (b, 0, 0)),
            pl.BlockSpec((1, N, DIM), lambda b, i: (b, 0, 0)),
            pl.BlockSpec((DIM, D), lambda b, i: (0, 0)),
            pl.BlockSpec((1, D), lambda b, i: (0, 0)),
        ],
        out_specs=pl.BlockSpec((1, I_BLK, D), lambda b, i: (b, i, 0)),
        out_shape=jax.ShapeDtypeStruct((B, N, D), jnp.float32),
    )(q3, counts, k3, v3, Wo, bo.reshape(1, D))
    return out
